# baseline (speedup 1.0000x reference)
import functools

import jax
import jax.numpy as jnp
from jax import lax
from jax.experimental import pallas as pl
from jax.experimental.pallas import tpu as pltpu
from jax.experimental.pallas import tpu_sc as plsc

_RES = (16, 32, 32, 32)
_C = 16
_B = 16384
_NPAIR = _RES[0] * _RES[1] * _RES[2] * (_RES[3] // 2)
_PSTR = (_RES[1] * _RES[2] * (_RES[3] // 2),
         _RES[2] * (_RES[3] // 2),
         _RES[3] // 2)
_NC = 2
_NS = 16
_NW = _NC * _NS
_PTS = _B // _NW
_CHUNK = 16
_NCHUNK = _PTS // _CHUNK
_NGRP = 64
_HGRP = _NGRP // 2
_RPG = 3
_NWPG = 5
_HIDX = _HGRP * _RPG * _CHUNK


def _sc_body(u_hbm, g_hbm, out_hbm, u_v, wbuf, idxbuf, rows_x, rows_y,
             outbuf, sem_x, sem_y):
    wid = lax.axis_index("s") * _NC + lax.axis_index("c")
    base_pt = wid * _PTS
    for d in range(4):
        pltpu.sync_copy(u_hbm.at[d, pl.ds(base_pt, _PTS)], u_v.at[d])

    def basis(cidx, d, n):
        uu = u_v[d, pl.ds(cidx * _CHUNK, _CHUNK)]
        x = jnp.clip(uu, 0.0, 1.0) * (n - 1)
        i = jnp.minimum(x.astype(jnp.int32), n - 2)
        t = x - i.astype(jnp.float32)
        t2 = t * t
        t3 = t2 * t
        sixth = jnp.float32(1.0 / 6.0)
        w0 = (-t3 + 3.0 * t2 - 3.0 * t + 1.0) * sixth
        w1 = (3.0 * t3 - 6.0 * t2 + 4.0) * sixth
        w2 = (-3.0 * t3 + 3.0 * t2 + 3.0 * t + 1.0) * sixth
        w3 = t3 * sixth
        lo = i == 0
        hi = i == n - 2
        zero = jnp.zeros_like(w0)
        W0 = jnp.where(lo, 2.0 * w0 + w1, jnp.where(hi, zero, w0))
        W1 = jnp.where(lo, w2 - w0, jnp.where(hi, w0, w1))
        W2 = jnp.where(lo, w3, jnp.where(hi, w1 - w3, w2))
        W3 = jnp.where(lo, zero, jnp.where(hi, w2 + 2.0 * w3, w3))
        base = jnp.clip(i - 1, 0, n - 4)
        return base, (W0, W1, W2, W3)

    def gen(cidx, par):
        W = []
        base_sum = None
        for d, n in enumerate(_RES[:3]):
            base, Wd = basis(cidx, d, n)
            W.append(Wd)
            contrib = base * _PSTR[d]
            base_sum = contrib if base_sum is None else base_sum + contrib
        bw, (V0, V1, V2, V3) = basis(cidx, 3, _RES[3])
        a = lax.shift_right_logical(bw, 1)
        even = (bw & 1) == 0
        zero = jnp.zeros_like(V0)
        W6 = (jnp.where(even, V0, zero),
              jnp.where(even, V1, V0),
              jnp.where(even, V2, V1),
              jnp.where(even, V3, V2),
              jnp.where(even, zero, V3))
        base_sum = base_sum + a

        g = 0
        for i in range(4):
            for j in range(4):
                s_ij = W[0][i] * W[1][j]
                for k in range(4):
                    s_ijk = s_ij * W[2][k]
                    goff = i * _PSTR[0] + j * _PSTR[1] + k * _PSTR[2]
                    gl = g % _HGRP
                    h = g // _HGRP
                    r0 = base_sum + goff
                    idxbuf[par, h, pl.ds(gl * _RPG * _CHUNK, _CHUNK)] = r0
                    idxbuf[par, h, pl.ds((gl * _RPG + 1) * _CHUNK, _CHUNK)] = (
                        r0 + 1)
                    idxbuf[par, h, pl.ds((gl * _RPG + 2) * _CHUNK, _CHUNK)] = (
                        jnp.minimum(r0 + 2, _NPAIR - 1))
                    for m in range(_NWPG):
                        wbuf[par, g * _NWPG + m, :] = s_ijk * W6[m]
                    g += 1

    def fire_half(par, h, rows_ref, sem):
        pltpu.async_copy(g_hbm.at[idxbuf.at[par, h]], rows_ref, sem)

    def wait_half(par, h, rows_ref, sem):
        pltpu.make_async_copy(
            g_hbm.at[idxbuf.at[par, h]], rows_ref, sem).wait()

    def compute_half(par, h, rows_ref, accs):
        def g_body(gl, accs):
            wrow = (h * _HGRP + gl) * _NWPG
            wv0 = wbuf[par, wrow, :]
            wv1 = wbuf[par, wrow + 1, :]
            wv2 = wbuf[par, wrow + 2, :]
            wv3 = wbuf[par, wrow + 3, :]
            wv4 = wbuf[par, wrow + 4, :]
            rbase = gl * _RPG * _CHUNK
            mask = jnp.int32(-65536)

            def cells(row):
                lo = plsc.bitcast(lax.shift_left(row, 16), jnp.float32)
                hi = plsc.bitcast(row & mask, jnp.float32)
                return lo, hi

            out = []
            for p in range(_CHUNK):
                c0, c1 = cells(rows_ref[rbase + p, :])
                c2, c3 = cells(rows_ref[rbase + _CHUNK + p, :])
                r2 = rows_ref[rbase + 2 * _CHUNK + p, :]
                c4 = plsc.bitcast(lax.shift_left(r2, 16), jnp.float32)
                acc = accs[p] + wv0[p] * c0 + wv1[p] * c1
                acc = acc + wv2[p] * c2 + wv3[p] * c3 + wv4[p] * c4
                out.append(acc)
            return tuple(out)

        return lax.fori_loop(0, _HGRP, g_body, accs)

    gen(0, 0)
    fire_half(0, 0, rows_x, sem_x)
    fire_half(0, 1, rows_y, sem_y)

    def chunk_body(c, carry):
        par = lax.rem(c, 2)
        nxt_par = 1 - par
        accs = tuple(jnp.zeros((_C,), jnp.float32) for _ in range(_CHUNK))
        wait_half(par, 0, rows_x, sem_x)
        accs = compute_half(par, 0, rows_x, accs)

        @pl.when(c + 1 < _NCHUNK)
        def _():
            gen(c + 1, nxt_par)
            fire_half(nxt_par, 0, rows_x, sem_x)

        wait_half(par, 1, rows_y, sem_y)
        accs = compute_half(par, 1, rows_y, accs)
        for p in range(_CHUNK):
            outbuf[c * _CHUNK + p, :] = accs[p]

        @pl.when(c + 1 < _NCHUNK)
        def _():
            fire_half(nxt_par, 1, rows_y, sem_y)

        return carry

    lax.fori_loop(0, _NCHUNK, chunk_body, 0)
    pltpu.sync_copy(outbuf, out_hbm.at[pl.ds(base_pt, _PTS)])


@functools.partial(
    pl.kernel,
    out_type=jax.ShapeDtypeStruct((_B, _C), jnp.float32),
    mesh=plsc.VectorSubcoreMesh(core_axis_name="c", subcore_axis_name="s"),
    scratch_types=[
        pltpu.VMEM((4, _PTS), jnp.float32),
        pltpu.VMEM((2, _NGRP * _NWPG, _CHUNK), jnp.float32),
        pltpu.VMEM((2, 2, _HIDX), jnp.int32),
        pltpu.VMEM((_HIDX, _C), jnp.int32),
        pltpu.VMEM((_HIDX, _C), jnp.int32),
        pltpu.VMEM((_PTS, _C), jnp.float32),
        pltpu.SemaphoreType.DMA,
        pltpu.SemaphoreType.DMA,
    ],
    compiler_params=pltpu.CompilerParams(use_tc_tiling_on_sc=False, needs_layout_passes=False),
)
def _interp_sc(u_hbm, g_hbm, out_hbm, u_v, wbuf, idxbuf, rows_x, rows_y,
               outbuf, sem_x, sem_y):
    _sc_body(u_hbm, g_hbm, out_hbm, u_v, wbuf, idxbuf, rows_x, rows_y,
             outbuf, sem_x, sem_y)


def kernel(u, grid):
    u_t = u.T
    g_pairs = (grid.astype(jnp.bfloat16)
               .reshape(_NPAIR, 2, _C)
               .swapaxes(-2, -1))
    g_i32 = jax.lax.bitcast_convert_type(g_pairs, jnp.int32)
    return _interp_sc(u_t, g_i32)


# speedup vs baseline: 3.0064x; 3.0064x over previous
import functools

import jax
import jax.numpy as jnp
from jax import lax
from jax.experimental import pallas as pl
from jax.experimental.pallas import tpu as pltpu
from jax.experimental.pallas import tpu_sc as plsc

_RES = (16, 32, 32, 32)
_C = 16
_B = 16384
_STR = (_RES[1] * _RES[2] * _RES[3], _RES[2] * _RES[3], _RES[3], 1)
_NROWS = _RES[0] * _RES[1] * _RES[2] * _RES[3]
_NC = 2
_NS = 16
_NW = _NC * _NS
_PTS = _B // _NW
_CHUNK = 16
_NCHUNK = _PTS // _CHUNK
_TAPS = 256
_HALF = _TAPS // 2
_IDXW = 128
_SLICES = _HALF * _CHUNK // _IDXW


def _sc_body(u_hbm, g_hbm, out_hbm, u_v, wbuf, idxbuf, rows_x, rows_y,
             outbuf, sem_x, sem_y):
    wid = lax.axis_index("s") * _NC + lax.axis_index("c")
    base_pt = wid * _PTS
    for d in range(4):
        pltpu.sync_copy(u_hbm.at[d, pl.ds(base_pt, _PTS)], u_v.at[d])

    def gen(cidx, par):
        W = []
        base_sum = None
        for d, n in enumerate(_RES):
            uu = u_v[d, pl.ds(cidx * _CHUNK, _CHUNK)]
            x = jnp.clip(uu, 0.0, 1.0) * (n - 1)
            i = jnp.minimum(x.astype(jnp.int32), n - 2)
            t = x - i.astype(jnp.float32)
            t2 = t * t
            t3 = t2 * t
            sixth = jnp.float32(1.0 / 6.0)
            w0 = (-t3 + 3.0 * t2 - 3.0 * t + 1.0) * sixth
            w1 = (3.0 * t3 - 6.0 * t2 + 4.0) * sixth
            w2 = (-3.0 * t3 + 3.0 * t2 + 3.0 * t + 1.0) * sixth
            w3 = t3 * sixth
            lo = i == 0
            hi = i == n - 2
            zero = jnp.zeros_like(w0)
            W0 = jnp.where(lo, 2.0 * w0 + w1, jnp.where(hi, zero, w0))
            W1 = jnp.where(lo, w2 - w0, jnp.where(hi, w0, w1))
            W2 = jnp.where(lo, w3, jnp.where(hi, w1 - w3, w2))
            W3 = jnp.where(lo, zero, jnp.where(hi, w2 + 2.0 * w3, w3))
            W.append((W0, W1, W2, W3))
            contrib = jnp.clip(i - 1, 0, n - 4) * _STR[d]
            base_sum = contrib if base_sum is None else base_sum + contrib

        q = 0
        for i in range(4):
            for j in range(4):
                s_ij = W[0][i] * W[1][j]
                for k in range(4):
                    s_ijk = s_ij * W[2][k]
                    for l in range(4):
                        off = i * _STR[0] + j * _STR[1] + k * _STR[2] + l
                        idxbuf[par, q // 8, pl.ds((q % 8) * _CHUNK, _CHUNK)] = (
                            base_sum + off)
                        wbuf[par, q, :] = s_ijk * W[3][l]
                        q += 1

    def fire_half(par, h, rows_ref, sem):
        for s in range(_SLICES):
            pltpu.async_copy(
                g_hbm.at[idxbuf.at[par, h * _SLICES + s]],
                rows_ref.at[pl.ds(s * _IDXW, _IDXW)],
                sem)

    def wait_half(par, h, rows_ref, sem):
        for s in range(_SLICES):
            pltpu.make_async_copy(
                g_hbm.at[idxbuf.at[par, h * _SLICES + s]],
                rows_ref.at[pl.ds(s * _IDXW, _IDXW)],
                sem).wait()

    def compute_half(par, h, rows_ref, accs):
        def q_body(ql, accs):
            wv = wbuf[par, h * _HALF + ql, :]
            base_row = ql * _CHUNK
            return tuple(
                accs[p] + wv[p] * rows_ref[base_row + p, :]
                for p in range(_CHUNK)
            )

        return lax.fori_loop(0, _HALF, q_body, accs, unroll=2)

    gen(0, 0)
    fire_half(0, 0, rows_x, sem_x)
    fire_half(0, 1, rows_y, sem_y)

    def chunk_body(c, carry):
        par = lax.rem(c, 2)
        nxt_par = 1 - par
        accs = tuple(jnp.zeros((_C,), jnp.float32) for _ in range(_CHUNK))
        wait_half(par, 0, rows_x, sem_x)
        accs = compute_half(par, 0, rows_x, accs)

        @pl.when(c + 1 < _NCHUNK)
        def _():
            gen(c + 1, nxt_par)
            fire_half(nxt_par, 0, rows_x, sem_x)

        wait_half(par, 1, rows_y, sem_y)
        accs = compute_half(par, 1, rows_y, accs)
        for p in range(_CHUNK):
            outbuf[c * _CHUNK + p, :] = accs[p]

        @pl.when(c + 1 < _NCHUNK)
        def _():
            fire_half(nxt_par, 1, rows_y, sem_y)

        return carry

    lax.fori_loop(0, _NCHUNK, chunk_body, 0)
    pltpu.sync_copy(outbuf, out_hbm.at[pl.ds(base_pt, _PTS)])


@functools.partial(
    pl.kernel,
    out_type=jax.ShapeDtypeStruct((_B, _C), jnp.float32),
    mesh=plsc.VectorSubcoreMesh(core_axis_name="c", subcore_axis_name="s"),
    scratch_types=[
        pltpu.VMEM((4, _PTS), jnp.float32),
        pltpu.VMEM((2, _TAPS, _CHUNK), jnp.float32),
        pltpu.VMEM((2, 2 * _SLICES, _IDXW), jnp.int32),
        pltpu.VMEM((_HALF * _CHUNK, _C), jnp.float32),
        pltpu.VMEM((_HALF * _CHUNK, _C), jnp.float32),
        pltpu.VMEM((_PTS, _C), jnp.float32),
        pltpu.SemaphoreType.DMA,
        pltpu.SemaphoreType.DMA,
    ],
    compiler_params=pltpu.CompilerParams(use_tc_tiling_on_sc=False),
)
def _interp_sc(u_hbm, g_hbm, out_hbm, u_v, wbuf, idxbuf, rows_x, rows_y,
               outbuf, sem_x, sem_y):
    _sc_body(u_hbm, g_hbm, out_hbm, u_v, wbuf, idxbuf, rows_x, rows_y,
             outbuf, sem_x, sem_y)


def kernel(u, grid):
    u_t = u.T
    return _interp_sc(u_t, grid.reshape(_NROWS, _C))
